# baseline (speedup 1.0000x reference)
import jax
import jax.numpy as jnp
from jax import lax
from jax.experimental import pallas as pl
from jax.experimental.pallas import tpu as pltpu
from jax.experimental.pallas import tpu_sc as plsc

_VOCAB = 100000
_EMB = 64
_B = 1024
_CTX = 20

_NC = 2
_NS = 16
_NW = _NC * _NS
_RPW = _B // _NW
_IPW = _RPW * _CTX
_CHUNK = 128
_NCHUNK = _IPW // _CHUNK

_LANES = 16
_KREG = _EMB // _LANES


def _gather_sum_body(idx_hbm, table_hbm, sums_hbm, idx_v, rows_v, sums_v, sem):
    wid = lax.axis_index("s") * _NC + lax.axis_index("c")
    pltpu.sync_copy(idx_hbm.at[wid], idx_v)
    copies = [
        pltpu.async_copy(
            table_hbm.at[idx_v.at[ch]],
            rows_v.at[pl.ds(ch * _CHUNK, _CHUNK)],
            sem,
        )
        for ch in range(_NCHUNK)
    ]
    for cp in copies:
        cp.wait()

    def row_body(r, carry):
        base = r * _CTX
        for k in range(_KREG):
            acc = rows_v[base, pl.ds(k * _LANES, _LANES)]
            for j in range(1, _CTX):
                acc = acc + rows_v[base + j, pl.ds(k * _LANES, _LANES)]
            sums_v[r, pl.ds(k * _LANES, _LANES)] = acc
        return carry

    lax.fori_loop(0, _RPW, row_body, 0)
    pltpu.sync_copy(sums_v, sums_hbm.at[pl.ds(wid * _RPW, _RPW)])


def _context_sums(inputs, emb_table):
    idx3 = inputs.reshape(_NW, _NCHUNK, _CHUNK)
    return pl.kernel(
        _gather_sum_body,
        out_type=jax.ShapeDtypeStruct((_B, _EMB), jnp.float32),
        mesh=plsc.VectorSubcoreMesh(core_axis_name="c", subcore_axis_name="s"),
        compiler_params=pltpu.CompilerParams(use_tc_tiling_on_sc=False),
        scratch_types=[
            pltpu.VMEM((_NCHUNK, _CHUNK), jnp.int32),
            pltpu.VMEM((_IPW, _EMB), jnp.float32),
            pltpu.VMEM((_RPW, _EMB), jnp.float32),
            pltpu.SemaphoreType.DMA,
        ],
    )(idx3, emb_table)


_VBLK = 2048
_NVB = (_VOCAB + _VBLK - 1) // _VBLK


def _proj_body(sums_ref, w_ref, b_ref, out_ref):
    out_ref[...] = (
        lax.dot_general(
            sums_ref[...], w_ref[...],
            (((1,), (1,)), ((), ())),
            preferred_element_type=jnp.float32,
        )
        + b_ref[...]
    )


def _project(sums, W, b):
    return pl.pallas_call(
        _proj_body,
        grid=(_NVB,),
        in_specs=[
            pl.BlockSpec((_B, _EMB), lambda j: (0, 0)),
            pl.BlockSpec((_VBLK, _EMB), lambda j: (j, 0)),
            pl.BlockSpec((1, _VBLK), lambda j: (0, j)),
        ],
        out_specs=pl.BlockSpec((_B, _VBLK), lambda j: (0, j)),
        out_shape=jax.ShapeDtypeStruct((_B, _VOCAB), jnp.float32),
        compiler_params=pltpu.CompilerParams(
            dimension_semantics=("arbitrary",),
        ),
    )(sums, W, b.reshape(1, _VOCAB))


def kernel(inputs, emb_table, W, b):
    sums = _context_sums(inputs, emb_table)
    return _project(sums, W, b)


# speedup vs baseline: 1.9140x; 1.9140x over previous
import jax
import jax.numpy as jnp
from jax import lax
from jax.experimental import pallas as pl
from jax.experimental.pallas import tpu as pltpu
from jax.experimental.pallas import tpu_sc as plsc

_VOCAB = 100000
_EMB = 64
_B = 1024
_CTX = 20

_NC = 2
_NS = 16
_NW = _NC * _NS
_RPW = _B // _NW
_IPW = _RPW * _CTX
_CHUNK = 128
_NCHUNK = _IPW // _CHUNK

_LANES = 16
_KREG = _EMB // _LANES


def _gather_sum_body(idx_hbm, table_hbm, sums_hbm, idx_v, rows_v, sums_v, sem):
    wid = lax.axis_index("s") * _NC + lax.axis_index("c")
    pltpu.sync_copy(idx_hbm.at[wid], idx_v)
    copies = [
        pltpu.async_copy(
            table_hbm.at[idx_v.at[ch]],
            rows_v.at[pl.ds(ch * _CHUNK, _CHUNK)],
            sem,
        )
        for ch in range(_NCHUNK)
    ]
    for cp in copies:
        cp.wait()

    def row_body(r, carry):
        base = r * _CTX
        for k in range(_KREG):
            acc = rows_v[base, pl.ds(k * _LANES, _LANES)]
            for j in range(1, _CTX):
                acc = acc + rows_v[base + j, pl.ds(k * _LANES, _LANES)]
            sums_v[r, pl.ds(k * _LANES, _LANES)] = acc
        return carry

    lax.fori_loop(0, _RPW, row_body, 0)
    pltpu.sync_copy(sums_v, sums_hbm.at[pl.ds(wid * _RPW, _RPW)])


def _context_sums(inputs, emb_table):
    idx3 = inputs.reshape(_NW, _NCHUNK, _CHUNK)
    return pl.kernel(
        _gather_sum_body,
        out_type=jax.ShapeDtypeStruct((_B, _EMB), jnp.float32),
        mesh=plsc.VectorSubcoreMesh(core_axis_name="c", subcore_axis_name="s"),
        compiler_params=pltpu.CompilerParams(use_tc_tiling_on_sc=False),
        scratch_types=[
            pltpu.VMEM((_NCHUNK, _CHUNK), jnp.int32),
            pltpu.VMEM((_IPW, _EMB), jnp.float32),
            pltpu.VMEM((_RPW, _EMB), jnp.float32),
            pltpu.SemaphoreType.DMA,
        ],
    )(idx3, emb_table)


_VBLK = 2048
_NVB = (_VOCAB + _VBLK - 1) // _VBLK


def _proj_body(w_ref, sums_ref, b_ref, out_ref):
    out_ref[...] = (
        lax.dot_general(
            w_ref[...], sums_ref[...],
            (((1,), (1,)), ((), ())),
            preferred_element_type=jnp.float32,
        )
        + b_ref[...]
    )


def _project(sums, W, b):
    out_t = pl.pallas_call(
        _proj_body,
        grid=(_NVB,),
        in_specs=[
            pl.BlockSpec((_VBLK, _EMB), lambda j: (j, 0)),
            pl.BlockSpec((_B, _EMB), lambda j: (0, 0)),
            pl.BlockSpec((_VBLK, 1), lambda j: (j, 0)),
        ],
        out_specs=pl.BlockSpec((_VBLK, _B), lambda j: (j, 0)),
        out_shape=jax.ShapeDtypeStruct((_VOCAB, _B), jnp.float32),
        compiler_params=pltpu.CompilerParams(
            dimension_semantics=("arbitrary",),
        ),
    )(W, sums, b.reshape(_VOCAB, 1))
    return out_t.T


def kernel(inputs, emb_table, W, b):
    sums = _context_sums(inputs, emb_table)
    return _project(sums, W, b)


# speedup vs baseline: 2.1444x; 1.1204x over previous
import jax
import jax.numpy as jnp
from jax import lax
from jax.experimental import pallas as pl
from jax.experimental.pallas import tpu as pltpu
from jax.experimental.pallas import tpu_sc as plsc

_VOCAB = 100000
_EMB = 64
_B = 1024
_CTX = 20

_NC = 2
_NS = 16
_NW = _NC * _NS
_RPW = _B // _NW
_IPW = _RPW * _CTX
_CHUNK = 128
_NCHUNK = _IPW // _CHUNK

_LANES = 16
_KREG = _EMB // _LANES


def _gather_sum_body(idx_hbm, table_hbm, sums_hbm, idx_v, rows_v, sums_v, sem):
    wid = lax.axis_index("s") * _NC + lax.axis_index("c")
    pltpu.sync_copy(idx_hbm.at[wid], idx_v)
    copies = [
        pltpu.async_copy(
            table_hbm.at[idx_v.at[ch]],
            rows_v.at[pl.ds(ch * _CHUNK, _CHUNK)],
            sem,
        )
        for ch in range(_NCHUNK)
    ]
    for cp in copies:
        cp.wait()

    def row_body(r, carry):
        base = r * _CTX
        for k in range(_KREG):
            acc = rows_v[base, pl.ds(k * _LANES, _LANES)]
            for j in range(1, _CTX):
                acc = acc + rows_v[base + j, pl.ds(k * _LANES, _LANES)]
            sums_v[r, pl.ds(k * _LANES, _LANES)] = acc
        return carry

    lax.fori_loop(0, _RPW, row_body, 0)
    pltpu.sync_copy(sums_v, sums_hbm.at[pl.ds(wid * _RPW, _RPW)])


def _context_sums(inputs, emb_table):
    idx3 = inputs.reshape(_NW, _NCHUNK, _CHUNK)
    return pl.kernel(
        _gather_sum_body,
        out_type=jax.ShapeDtypeStruct((_B, _EMB), jnp.float32),
        mesh=plsc.VectorSubcoreMesh(core_axis_name="c", subcore_axis_name="s"),
        compiler_params=pltpu.CompilerParams(use_tc_tiling_on_sc=False),
        scratch_types=[
            pltpu.VMEM((_NCHUNK, _CHUNK), jnp.int32),
            pltpu.VMEM((_IPW, _EMB), jnp.float32),
            pltpu.VMEM((_RPW, _EMB), jnp.float32),
            pltpu.SemaphoreType.DMA,
        ],
    )(idx3, emb_table)


_VBLK = 2048
_NVB = (_VOCAB + _VBLK - 1) // _VBLK


def _proj_body(wt_ref, sums_ref, b_ref, out_ref):
    out_ref[...] = (
        lax.dot_general(
            wt_ref[...], sums_ref[...],
            (((0,), (1,)), ((), ())),
            preferred_element_type=jnp.float32,
        )
        + b_ref[0]
    )


def _project(sums, W, b):
    bcol = jnp.pad(b, (0, _NVB * _VBLK - _VOCAB)).reshape(_NVB, _VBLK, 1)
    out_t = pl.pallas_call(
        _proj_body,
        grid=(_NVB,),
        in_specs=[
            pl.BlockSpec((_EMB, _VBLK), lambda j: (0, j)),
            pl.BlockSpec((_B, _EMB), lambda j: (0, 0)),
            pl.BlockSpec((1, _VBLK, 1), lambda j: (j, 0, 0)),
        ],
        out_specs=pl.BlockSpec((_VBLK, _B), lambda j: (j, 0)),
        out_shape=jax.ShapeDtypeStruct((_VOCAB, _B), jnp.float32),
        compiler_params=pltpu.CompilerParams(
            dimension_semantics=("arbitrary",),
        ),
    )(W.T, sums, bcol)
    return out_t.T


def kernel(inputs, emb_table, W, b):
    sums = _context_sums(inputs, emb_table)
    return _project(sums, W, b)


# speedup vs baseline: 2.7450x; 1.2801x over previous
import jax
import jax.numpy as jnp
from jax import lax
from jax.experimental import pallas as pl
from jax.experimental.pallas import tpu as pltpu
from jax.experimental.pallas import tpu_sc as plsc

_VOCAB = 100000
_EMB = 64
_B = 1024
_CTX = 20

_NC = 2
_NS = 16
_NW = _NC * _NS
_RPW = _B // _NW
_IPW = _RPW * _CTX
_CHUNK = 128
_NCHUNK = _IPW // _CHUNK

_LANES = 16
_KREG = _EMB // _LANES


def _gather_sum_body(idx_hbm, table_hbm, sums_hbm, idx_v, rows_v, sums_v, sem):
    wid = lax.axis_index("s") * _NC + lax.axis_index("c")
    pltpu.sync_copy(idx_hbm.at[wid], idx_v)
    copies = [
        pltpu.async_copy(
            table_hbm.at[idx_v.at[ch]],
            rows_v.at[pl.ds(ch * _CHUNK, _CHUNK)],
            sem,
        )
        for ch in range(_NCHUNK)
    ]
    for cp in copies:
        cp.wait()

    def row_body(r, carry):
        base = r * _CTX
        for k in range(_KREG):
            acc = rows_v[base, pl.ds(k * _LANES, _LANES)]
            for j in range(1, _CTX):
                acc = acc + rows_v[base + j, pl.ds(k * _LANES, _LANES)]
            sums_v[r, pl.ds(k * _LANES, _LANES)] = acc
        return carry

    lax.fori_loop(0, _RPW, row_body, 0)
    pltpu.sync_copy(sums_v, sums_hbm.at[pl.ds(wid * _RPW, _RPW)])


def _context_sums(inputs, emb_table):
    idx3 = inputs.reshape(_NW, _NCHUNK, _CHUNK)
    return pl.kernel(
        _gather_sum_body,
        out_type=jax.ShapeDtypeStruct((_B, _EMB), jnp.float32),
        mesh=plsc.VectorSubcoreMesh(core_axis_name="c", subcore_axis_name="s"),
        compiler_params=pltpu.CompilerParams(use_tc_tiling_on_sc=False),
        scratch_types=[
            pltpu.VMEM((_NCHUNK, _CHUNK), jnp.int32),
            pltpu.VMEM((_IPW, _EMB), jnp.float32),
            pltpu.VMEM((_RPW, _EMB), jnp.float32),
            pltpu.SemaphoreType.DMA,
        ],
    )(idx3, emb_table)


_VBLK = 2048
_NVB = (_VOCAB + _VBLK - 1) // _VBLK


def _proj_body(wt_ref, sums_ref, b_ref, out_ref):
    lhs = jnp.concatenate([wt_ref[...], b_ref[...]], axis=0)
    rhs = jnp.concatenate(
        [sums_ref[...], jnp.ones((_B, 1), jnp.float32)], axis=1
    )
    out_ref[...] = lax.dot_general(
        lhs, rhs,
        (((0,), (1,)), ((), ())),
        preferred_element_type=jnp.float32,
    )


def _project(sums, W, b):
    out_t = pl.pallas_call(
        _proj_body,
        grid=(_NVB,),
        in_specs=[
            pl.BlockSpec((_EMB, _VBLK), lambda j: (0, j)),
            pl.BlockSpec((_B, _EMB), lambda j: (0, 0)),
            pl.BlockSpec((1, _VBLK), lambda j: (0, j)),
        ],
        out_specs=pl.BlockSpec((_VBLK, _B), lambda j: (j, 0)),
        out_shape=jax.ShapeDtypeStruct((_VOCAB, _B), jnp.float32),
        compiler_params=pltpu.CompilerParams(
            dimension_semantics=("arbitrary",),
        ),
    )(W.T, sums, b.reshape(1, _VOCAB))
    return out_t.T


def kernel(inputs, emb_table, W, b):
    sums = _context_sums(inputs, emb_table)
    return _project(sums, W, b)


# speedup vs baseline: 2.7751x; 1.0110x over previous
import jax
import jax.numpy as jnp
from jax import lax
from jax.experimental import pallas as pl
from jax.experimental.pallas import tpu as pltpu
from jax.experimental.pallas import tpu_sc as plsc

_VOCAB = 100000
_EMB = 64
_B = 1024
_CTX = 20

_NC = 2
_NS = 16
_NW = _NC * _NS
_RPW = _B // _NW
_IPW = _RPW * _CTX
_CHUNK = 128
_NCHUNK = _IPW // _CHUNK

_LANES = 16
_KREG = _EMB // _LANES


def _gather_sum_body(idx_hbm, table_hbm, sums_hbm, idx_v, rows_v, sums_v, sem):
    wid = lax.axis_index("s") * _NC + lax.axis_index("c")
    pltpu.sync_copy(idx_hbm.at[wid], idx_v)
    copies = [
        pltpu.async_copy(
            table_hbm.at[idx_v.at[ch]],
            rows_v.at[pl.ds(ch * _CHUNK, _CHUNK)],
            sem,
        )
        for ch in range(_NCHUNK)
    ]
    for cp in copies:
        cp.wait()

    def row_body(r, carry):
        base = r * _CTX
        for k in range(_KREG):
            acc = rows_v[base, pl.ds(k * _LANES, _LANES)]
            for j in range(1, _CTX):
                acc = acc + rows_v[base + j, pl.ds(k * _LANES, _LANES)]
            sums_v[r, pl.ds(k * _LANES, _LANES)] = acc
        return carry

    lax.fori_loop(0, _RPW, row_body, 0)
    pltpu.sync_copy(sums_v, sums_hbm.at[pl.ds(wid * _RPW, _RPW)])


def _context_sums(inputs, emb_table):
    idx3 = inputs.reshape(_NW, _NCHUNK, _CHUNK)
    return pl.kernel(
        _gather_sum_body,
        out_type=jax.ShapeDtypeStruct((_B, _EMB), jnp.float32),
        mesh=plsc.VectorSubcoreMesh(core_axis_name="c", subcore_axis_name="s"),
        compiler_params=pltpu.CompilerParams(use_tc_tiling_on_sc=False),
        scratch_types=[
            pltpu.VMEM((_NCHUNK, _CHUNK), jnp.int32),
            pltpu.VMEM((_IPW, _EMB), jnp.float32),
            pltpu.VMEM((_RPW, _EMB), jnp.float32),
            pltpu.SemaphoreType.DMA,
        ],
    )(idx3, emb_table)


_VBLK = 4096
_NVB = (_VOCAB + _VBLK - 1) // _VBLK


def _proj_body(wt_ref, sums_ref, b_ref, out_ref):
    lhs = jnp.concatenate([wt_ref[...], b_ref[...]], axis=0)
    rhs = jnp.concatenate(
        [sums_ref[...], jnp.ones((_B, 1), jnp.float32)], axis=1
    )
    out_ref[...] = lax.dot_general(
        lhs, rhs,
        (((0,), (1,)), ((), ())),
        preferred_element_type=jnp.float32,
    )


def _project(sums, W, b):
    out_t = pl.pallas_call(
        _proj_body,
        grid=(_NVB,),
        in_specs=[
            pl.BlockSpec((_EMB, _VBLK), lambda j: (0, j)),
            pl.BlockSpec((_B, _EMB), lambda j: (0, 0)),
            pl.BlockSpec((1, _VBLK), lambda j: (0, j)),
        ],
        out_specs=pl.BlockSpec((_VBLK, _B), lambda j: (j, 0)),
        out_shape=jax.ShapeDtypeStruct((_VOCAB, _B), jnp.float32),
        compiler_params=pltpu.CompilerParams(
            dimension_semantics=("arbitrary",),
        ),
    )(W.T, sums, b.reshape(1, _VOCAB))
    return out_t.T


def kernel(inputs, emb_table, W, b):
    sums = _context_sums(inputs, emb_table)
    return _project(sums, W, b)


# speedup vs baseline: 2.9840x; 1.0753x over previous
import jax
import jax.numpy as jnp
from jax import lax
from jax.experimental import pallas as pl
from jax.experimental.pallas import tpu as pltpu
from jax.experimental.pallas import tpu_sc as plsc

_VOCAB = 100000
_EMB = 64
_B = 1024
_CTX = 20

_NC = 2
_NS = 16
_NW = _NC * _NS
_LANES = 16
_NG = _B // _LANES


def _gather_sum_body(idxt_hbm, embt_hbm, sumst_hbm, idx_v, tab_v, col_v):
    wid = lax.axis_index("s") * _NC + lax.axis_index("c")
    pltpu.sync_copy(idxt_hbm, idx_v)
    for half in range(_EMB // _NW):
        k = wid + half * _NW
        pltpu.sync_copy(embt_hbm.at[k], tab_v)

        def group_body(g, carry):
            base = g * _LANES
            acc = jnp.zeros((_LANES,), jnp.float32)
            for j in range(_CTX):
                iv = idx_v[j, pl.ds(base, _LANES)]
                acc = acc + plsc.load_gather(tab_v, [iv])
            col_v[pl.ds(base, _LANES)] = acc
            return carry

        lax.fori_loop(0, _NG, group_body, 0)
        pltpu.sync_copy(col_v, sumst_hbm.at[k])


def _context_sums_t(inputs, emb_table):
    return pl.kernel(
        _gather_sum_body,
        out_type=jax.ShapeDtypeStruct((_EMB, _B), jnp.float32),
        mesh=plsc.VectorSubcoreMesh(core_axis_name="c", subcore_axis_name="s"),
        compiler_params=pltpu.CompilerParams(
            use_tc_tiling_on_sc=False, needs_layout_passes=False
        ),
        scratch_types=[
            pltpu.VMEM((_CTX, _B), jnp.int32),
            pltpu.VMEM((_VOCAB,), jnp.float32),
            pltpu.VMEM((_B,), jnp.float32),
        ],
    )(inputs.T, emb_table.T)


_VBLK = 4096
_NVB = (_VOCAB + _VBLK - 1) // _VBLK


def _proj_body(wt_ref, sumst_ref, b_ref, out_ref):
    lhs = jnp.concatenate([wt_ref[...], b_ref[...]], axis=0)
    rhs = jnp.concatenate(
        [sumst_ref[...], jnp.ones((1, _B), jnp.float32)], axis=0
    )
    out_ref[...] = lax.dot_general(
        lhs, rhs,
        (((0,), (0,)), ((), ())),
        preferred_element_type=jnp.float32,
    )


def _project(sums_t, W, b):
    out_t = pl.pallas_call(
        _proj_body,
        grid=(_NVB,),
        in_specs=[
            pl.BlockSpec((_EMB, _VBLK), lambda j: (0, j)),
            pl.BlockSpec((_EMB, _B), lambda j: (0, 0)),
            pl.BlockSpec((1, _VBLK), lambda j: (0, j)),
        ],
        out_specs=pl.BlockSpec((_VBLK, _B), lambda j: (j, 0)),
        out_shape=jax.ShapeDtypeStruct((_VOCAB, _B), jnp.float32),
        compiler_params=pltpu.CompilerParams(
            dimension_semantics=("arbitrary",),
        ),
    )(W.T, sums_t, b.reshape(1, _VOCAB))
    return out_t.T


def kernel(inputs, emb_table, W, b):
    sums_t = _context_sums_t(inputs, emb_table)
    return _project(sums_t, W, b)


# speedup vs baseline: 3.5959x; 1.2051x over previous
import jax
import jax.numpy as jnp
from jax import lax
from jax.experimental import pallas as pl
from jax.experimental.pallas import tpu as pltpu
from jax.experimental.pallas import tpu_sc as plsc

_VOCAB = 100000
_EMB = 64
_B = 1024
_CTX = 20

_NC = 2
_NS = 16
_NW = _NC * _NS
_LANES = 16
_NG = _B // _LANES


def _gather_sum_body(idxt_hbm, embt_hbm, sumst_hbm, idx_v, tab_v, col_v):
    wid = lax.axis_index("s") * _NC + lax.axis_index("c")
    pltpu.sync_copy(idxt_hbm, idx_v)
    for half in range(_EMB // _NW):
        k = wid + half * _NW
        pltpu.sync_copy(embt_hbm.at[k], tab_v)

        def group_body(g, carry):
            base = g * _LANES
            acc = jnp.zeros((_LANES,), jnp.float32)
            for j in range(_CTX):
                iv = idx_v[j, pl.ds(base, _LANES)]
                acc = acc + plsc.load_gather(tab_v, [iv])
            col_v[pl.ds(base, _LANES)] = acc
            return carry

        lax.fori_loop(0, _NG, group_body, 0)
        pltpu.sync_copy(col_v, sumst_hbm.at[k])


def _context_sums_t(inputs, emb_table):
    return pl.kernel(
        _gather_sum_body,
        out_type=jax.ShapeDtypeStruct((_EMB, _B), jnp.float32),
        mesh=plsc.VectorSubcoreMesh(core_axis_name="c", subcore_axis_name="s"),
        compiler_params=pltpu.CompilerParams(
            use_tc_tiling_on_sc=True, needs_layout_passes=False
        ),
        scratch_types=[
            pltpu.VMEM((_CTX, _B), jnp.int32),
            pltpu.VMEM((_VOCAB,), jnp.float32),
            pltpu.VMEM((_B,), jnp.float32),
        ],
    )(inputs.T, emb_table.T)


_VBLK = 4096
_NVB = (_VOCAB + _VBLK - 1) // _VBLK


def _proj_body(wt_ref, sumst_ref, b_ref, out_ref):
    lhs = jnp.concatenate([wt_ref[...], b_ref[...]], axis=0)
    rhs = jnp.concatenate(
        [sumst_ref[...], jnp.ones((1, _B), jnp.float32)], axis=0
    )
    out_ref[...] = lax.dot_general(
        lhs, rhs,
        (((0,), (0,)), ((), ())),
        preferred_element_type=jnp.float32,
    )


def _project(sums_t, W, b):
    out_t = pl.pallas_call(
        _proj_body,
        grid=(_NVB,),
        in_specs=[
            pl.BlockSpec((_EMB, _VBLK), lambda j: (0, j)),
            pl.BlockSpec((_EMB, _B), lambda j: (0, 0)),
            pl.BlockSpec((1, _VBLK), lambda j: (0, j)),
        ],
        out_specs=pl.BlockSpec((_VBLK, _B), lambda j: (j, 0)),
        out_shape=jax.ShapeDtypeStruct((_VOCAB, _B), jnp.float32),
        compiler_params=pltpu.CompilerParams(
            dimension_semantics=("arbitrary",),
        ),
    )(W.T, sums_t, b.reshape(1, _VOCAB))
    return out_t.T


def kernel(inputs, emb_table, W, b):
    sums_t = _context_sums_t(inputs, emb_table)
    return _project(sums_t, W, b)
